# baseline (speedup 1.0000x reference)
import functools

import jax
import jax.numpy as jnp
from jax import lax
from jax.experimental import pallas as pl
from jax.experimental.pallas import tpu as pltpu
from jax.experimental.pallas import tpu_sc as plsc

VOCAB = 1000000
EMBED_DIM = 32
BATCH = 16384
HIST = 200

_NC = 2
_NS = 16
_NW = _NC * _NS
_BW = BATCH // _NW
_CBW = _BW // 128
_TB = 200
_NB = VOCAB // _TB
_KFULL = _NB // _NS


def _transpose_block(src, dst, iota):
    hi_iota = iota + 16

    def body(k, c):
        for u in range(10):
            v = k * 10 + u
            col = jnp.full((16,), v, jnp.int32)
            dst[v, pl.ds(0, 16)] = plsc.load_gather(src, [iota, col])
            dst[v, pl.ds(16, 16)] = plsc.load_gather(src, [hi_iota, col])
        return c

    lax.fori_loop(0, _TB // 10, body, 0, unroll=False)


def _phase1(wt, wrm, src, dst, sr, sw, sid, iota):
    def read(k, b):
        v0 = (sid + k * _NS) * _TB
        pltpu.async_copy(wt.at[:, pl.ds(v0, _TB)], src[b], sr[b])

    def wait_read(k, b):
        v0 = (sid + k * _NS) * _TB
        pltpu.make_async_copy(wt.at[:, pl.ds(v0, _TB)], src[b], sr[b]).wait()

    def write(k, b):
        v0 = (sid + k * _NS) * _TB
        pltpu.async_copy(dst[b], wrm.at[pl.ds(v0, _TB), :], sw[b])

    def wait_write(k, b):
        v0 = (sid + k * _NS) * _TB
        pltpu.make_async_copy(dst[b], wrm.at[pl.ds(v0, _TB), :], sw[b]).wait()

    for b in range(2):
        read(b, b)
    for b in range(2):
        wait_read(b, b)
        _transpose_block(src[b], dst[b], iota)
        write(b, b)
        read(b + 2, b)

    def steady(g, c):
        for b in range(2):
            k = g * 2 + b
            wait_read(k, b)
            wait_write(k - 2, b)
            _transpose_block(src[b], dst[b], iota)
            write(k, b)
            read(k + 2, b)
        return c

    lax.fori_loop(1, _KFULL // 2 - 1, steady, 0, unroll=False)

    for b in range(2):
        k = _KFULL - 2 + b
        wait_read(k, b)
        wait_write(k - 2, b)
        _transpose_block(src[b], dst[b], iota)
        write(k, b)
    for b in range(2):
        wait_write(_KFULL - 2 + b, b)

    @pl.when(sid < _NB - _KFULL * _NS)
    def _():
        read(_KFULL, 0)
        wait_read(_KFULL, 0)
        _transpose_block(src[0], dst[0], iota)
        write(_KFULL, 0)
        wait_write(_KFULL, 0)


def _transpose_chunk(rows, tbuf, iota):
    def body(d, c):
        rd = d // 8
        sd = d % 8
        col = jnp.full((16,), d, jnp.int32)
        for cb in range(4):
            for s in range(8):
                rids = iota + (cb * 128 + 16 * s)
                tbuf[rd, cb, sd, pl.ds(16 * s, 16)] = plsc.load_gather(
                    rows, [rids, col])
        return c

    lax.fori_loop(0, 32, body, 0, unroll=False)


def _phase2(idx_hbm, tbl, out5, idx_v, rows, tbuf, si, sg, sw, wid, iota):
    b0 = wid * _BW
    cb0 = wid * _CBW

    def gather(b):
        pltpu.async_copy(tbl.at[idx_v[b]], rows[b], sg[b])

    def wait_gather(b):
        pltpu.make_async_copy(tbl.at[idx_v[b]], rows[b], sg[b]).wait()

    def write(i, b):
        pltpu.async_copy(tbuf[b], out5.at[i, :, pl.ds(cb0, _CBW), :, :], sw[b])

    def wait_write(i, b):
        pltpu.make_async_copy(
            tbuf[b], out5.at[i, :, pl.ds(cb0, _CBW), :, :], sw[b]).wait()

    def load_idx(i, b):
        pltpu.async_copy(idx_hbm.at[pl.ds(i * BATCH + b0, _BW)], idx_v[b],
                         si[b])

    def wait_idx(i, b):
        pltpu.make_async_copy(idx_hbm.at[pl.ds(i * BATCH + b0, _BW)], idx_v[b],
                              si[b]).wait()

    for b in range(2):
        load_idx(b, b)
    for b in range(2):
        wait_idx(b, b)
        gather(b)
    for b in range(2):
        wait_gather(b)
        load_idx(b + 2, b)
        _transpose_chunk(rows[b], tbuf[b], iota)
        write(b, b)
        wait_idx(b + 2, b)
        gather(b)

    def steady(g, c):
        for b in range(2):
            i = g * 2 + b
            wait_gather(b)
            load_idx(i + 2, b)
            wait_write(i - 2, b)
            _transpose_chunk(rows[b], tbuf[b], iota)
            write(i, b)
            wait_idx(i + 2, b)
            gather(b)
        return c

    lax.fori_loop(1, HIST // 2 - 1, steady, 0, unroll=False)

    for b in range(2):
        i = HIST - 2 + b
        wait_gather(b)
        wait_write(i - 2, b)
        _transpose_chunk(rows[b], tbuf[b], iota)
        write(i, b)
    for b in range(2):
        wait_write(HIST - 2 + b, b)


def _embed_body(idx_hbm, wt_hbm, out5, wrm_a, wrm_b,
                p1s0, p1s1, p1d0, p1d1,
                iv0, iv1, rv0, rv1, tb0, tb1,
                sr0, sr1, sw0, sw1,
                si0, si1, sg0, sg1, so0, so1):
    cid = lax.axis_index("c")
    sid = lax.axis_index("s")
    iota = lax.iota(jnp.int32, 16)
    p1src = [p1s0, p1s1]
    p1dst = [p1d0, p1d1]
    sr = [sr0, sr1]
    sw = [sw0, sw1]
    idx_v = [iv0, iv1]
    rows = [rv0, rv1]
    tbuf = [tb0, tb1]
    si = [si0, si1]
    sg = [sg0, sg1]
    so = [so0, so1]

    @pl.when(cid == 0)
    def _():
        _phase1(wt_hbm, wrm_a, p1src, p1dst, sr, sw, sid, iota)

    @pl.when(cid == 1)
    def _():
        _phase1(wt_hbm, wrm_b, p1src, p1dst, sr, sw, sid, iota)

    plsc.subcore_barrier()

    @pl.when(cid == 0)
    def _():
        _phase2(idx_hbm, wrm_a, out5, idx_v, rows, tbuf, si, sg, so,
                sid * _NC, iota)

    @pl.when(cid == 1)
    def _():
        _phase2(idx_hbm, wrm_b, out5, idx_v, rows, tbuf, si, sg, so,
                sid * _NC + 1, iota)


@jax.jit
def _embed(idx, wt):
    fn = pl.kernel(
        _embed_body,
        mesh=plsc.VectorSubcoreMesh(core_axis_name="c", subcore_axis_name="s"),
        out_type=[
            jax.ShapeDtypeStruct((HIST, 4, 128, 8, 128), jnp.float32),
            jax.ShapeDtypeStruct((VOCAB, EMBED_DIM), jnp.float32),
            jax.ShapeDtypeStruct((VOCAB, EMBED_DIM), jnp.float32),
        ],
        scratch_types=(
            [pltpu.VMEM((EMBED_DIM, _TB), jnp.float32) for _ in range(2)]
            + [pltpu.VMEM((_TB, EMBED_DIM), jnp.float32) for _ in range(2)]
            + [pltpu.VMEM((_BW,), jnp.int32) for _ in range(2)]
            + [pltpu.VMEM((_BW, EMBED_DIM), jnp.float32) for _ in range(2)]
            + [pltpu.VMEM((4, _CBW, 8, 128), jnp.float32) for _ in range(2)]
            + [pltpu.SemaphoreType.DMA for _ in range(10)]
        ),
        compiler_params=pltpu.CompilerParams(use_tc_tiling_on_sc=False,
                                             needs_layout_passes=False),
    )
    return fn(idx, wt)


def kernel(x, weight):
    idx = x.T.reshape(-1).astype(jnp.int32)
    out5, _, _ = _embed(idx, weight.T)
    return out5.transpose(2, 4, 0, 1, 3).reshape(BATCH, HIST, EMBED_DIM)


# speedup vs baseline: 3.1071x; 3.1071x over previous
import jax
import jax.numpy as jnp
from jax import lax
from jax.experimental import pallas as pl
from jax.experimental.pallas import tpu as pltpu
from jax.experimental.pallas import tpu_sc as plsc

VOCAB = 1000000
EMBED_DIM = 32
BATCH = 16384
HIST = 200

_NC = 2
_NS = 16
_NW = _NC * _NS
_BW = BATCH // _NW
_CBW = _BW // 128


def _transpose_chunk(rows, tbuf, iota):
    @plsc.parallel_loop(0, EMBED_DIM, unroll=2)
    def _(d):
        rd = d // 8
        sd = d % 8
        col = jnp.full((16,), d, jnp.int32)
        for cb in range(4):
            for s in range(8):
                rids = iota + (cb * 128 + 16 * s)
                tbuf[rd, cb, sd, pl.ds(16 * s, 16)] = plsc.load_gather(
                    rows, [rids, col])


def _embed_body(idx_hbm, tbl_hbm, out5,
                iv0, iv1, rv0, rv1, tb0, tb1,
                si0, si1, sg0, sg1, so0, so1):
    cid = lax.axis_index("c")
    sid = lax.axis_index("s")
    iota = lax.iota(jnp.int32, 16)
    idx_v = [iv0, iv1]
    rows = [rv0, rv1]
    tbuf = [tb0, tb1]
    si = [si0, si1]
    sg = [sg0, sg1]
    so = [so0, so1]

    wid = sid * _NC + cid
    b0 = wid * _BW
    cb0 = wid * _CBW

    def gather(b):
        pltpu.async_copy(tbl_hbm.at[idx_v[b]], rows[b], sg[b])

    def wait_gather(b):
        pltpu.make_async_copy(tbl_hbm.at[idx_v[b]], rows[b], sg[b]).wait()

    def write(i, b):
        pltpu.async_copy(tbuf[b], out5.at[i, :, pl.ds(cb0, _CBW), :, :],
                         so[b])

    def wait_write(i, b):
        pltpu.make_async_copy(
            tbuf[b], out5.at[i, :, pl.ds(cb0, _CBW), :, :], so[b]).wait()

    def load_idx(i, b):
        pltpu.async_copy(idx_hbm.at[pl.ds(i * BATCH + b0, _BW)], idx_v[b],
                         si[b])

    def wait_idx(i, b):
        pltpu.make_async_copy(idx_hbm.at[pl.ds(i * BATCH + b0, _BW)],
                              idx_v[b], si[b]).wait()

    for b in range(2):
        load_idx(b, b)
    for b in range(2):
        wait_idx(b, b)
        gather(b)
    for b in range(2):
        wait_gather(b)
        load_idx(b + 2, b)
        _transpose_chunk(rows[b], tbuf[b], iota)
        write(b, b)
        wait_idx(b + 2, b)
        gather(b)

    def steady(g, c):
        for b in range(2):
            i = g * 2 + b
            wait_gather(b)
            load_idx(i + 2, b)
            wait_write(i - 2, b)
            _transpose_chunk(rows[b], tbuf[b], iota)
            write(i, b)
            wait_idx(i + 2, b)
            gather(b)
        return c

    lax.fori_loop(1, HIST // 2 - 1, steady, 0, unroll=False)

    for b in range(2):
        i = HIST - 2 + b
        wait_gather(b)
        wait_write(i - 2, b)
        _transpose_chunk(rows[b], tbuf[b], iota)
        write(i, b)
    for b in range(2):
        wait_write(HIST - 2 + b, b)


@jax.jit
def _embed(idx, tbl):
    fn = pl.kernel(
        _embed_body,
        mesh=plsc.VectorSubcoreMesh(core_axis_name="c", subcore_axis_name="s"),
        out_type=jax.ShapeDtypeStruct((HIST, 4, 128, 8, 128), jnp.float32),
        scratch_types=(
            [pltpu.VMEM((_BW,), jnp.int32) for _ in range(2)]
            + [pltpu.VMEM((_BW, EMBED_DIM), jnp.float32) for _ in range(2)]
            + [pltpu.VMEM((4, _CBW, 8, 128), jnp.float32) for _ in range(2)]
            + [pltpu.SemaphoreType.DMA for _ in range(6)]
        ),
        compiler_params=pltpu.CompilerParams(use_tc_tiling_on_sc=False,
                                             needs_layout_passes=False),
    )
    return fn(idx, tbl)


def kernel(x, weight):
    idx = x.T.reshape(-1).astype(jnp.int32)
    out5 = _embed(idx, weight)
    return out5.transpose(2, 4, 0, 1, 3).reshape(BATCH, HIST, EMBED_DIM)
